# Initial kernel scaffold; baseline (speedup 1.0000x reference)
import jax
import jax.numpy as jnp
from jax.experimental import pallas as pl


def kernel(x, edge_index, edge_weight, W1, b1, W2, b2, W3, b3):
    raise NotImplementedError("write your pallas kernel here")


# speedup vs baseline: 3.6285x; 3.6285x over previous
import dataclasses
import functools

import jax
import jax.numpy as jnp
from jax import lax
from jax.experimental import pallas as pl
from jax.experimental.pallas import tpu as pltpu
from jax.experimental.pallas import tpu_sc as plsc

N_NODES = 10000
N_EDGES = 320000
D = 128
NC = 2
NS = 16
NW = NC * NS
K = 128
NCHUNK = 79
SUB = 16
NSUB = K // SUB
EPT = NCHUNK * K
N_PAD = 10112
RK = 128
NRCHUNK = N_PAD // RK
RC_PER_TILE = 5
F16 = D // 16


def _spmm_sc(y, col3, row3, w3):
    mesh = plsc.VectorSubcoreMesh(core_axis_name="c", subcore_axis_name="s")
    cp = pltpu.CompilerParams()
    if "needs_layout_passes" in pltpu.CompilerParams.__dataclass_fields__:
        cp = dataclasses.replace(cp, needs_layout_passes=False)

    @functools.partial(
        pl.kernel,
        compiler_params=cp,
        out_type=jax.ShapeDtypeStruct((NC, N_PAD, D), jnp.float32),
        mesh=mesh,
        scratch_types=[
            pltpu.VMEM((NCHUNK, K), jnp.int32),
            pltpu.VMEM((NCHUNK, K), jnp.int32),
            pltpu.VMEM((NCHUNK, K), jnp.float32),
            pltpu.VMEM((K, D), jnp.float32),
            pltpu.VMEM_SHARED((N_PAD, D), jnp.float32),
            pltpu.SemaphoreType.DMA,
        ],
    )
    def spmm_kernel(y_hbm, col_hbm, row_hbm, w_hbm, out_hbm,
                    col_v, row_v, w_v, rows_v, acc, sem):
        cid = lax.axis_index("c")
        sid = lax.axis_index("s")
        wid = cid * NS + sid

        @pl.loop(0, RK)
        def _(r):
            for f in range(F16):
                rows_v[r, pl.ds(16 * f, 16)] = jnp.zeros((16,), jnp.float32)

        for i in range(RC_PER_TILE):
            ci = sid * RC_PER_TILE + i

            @pl.when(ci < NRCHUNK)
            def _():
                pltpu.sync_copy(rows_v, acc.at[pl.ds(ci * RK, RK)])

        pltpu.sync_copy(col_hbm.at[wid], col_v)
        pltpu.sync_copy(row_hbm.at[wid], row_v)
        pltpu.sync_copy(w_hbm.at[wid], w_v)

        plsc.subcore_barrier()

        @pl.loop(0, NCHUNK)
        def _(c):
            pltpu.async_copy(y_hbm.at[col_v.at[c]], rows_v, sem).wait()

            @pl.loop(0, K)
            def _(e):
                wsplat = plsc.load_gather(
                    w_v, [jnp.broadcast_to(c, (16,)), jnp.broadcast_to(e, (16,))])
                for f in range(F16):
                    sl = pl.ds(16 * f, 16)
                    rows_v[e, sl] = rows_v[e, sl] * wsplat

            for j in range(NSUB):
                idx16 = row_v[c, pl.ds(j * SUB, SUB)]
                pltpu.sync_copy(rows_v.at[pl.ds(j * SUB, SUB)],
                                acc.at[idx16], add=True)

        plsc.subcore_barrier()

        for i in range(RC_PER_TILE):
            ci = sid * RC_PER_TILE + i

            @pl.when(ci < NRCHUNK)
            def _():
                pltpu.sync_copy(acc.at[pl.ds(ci * RK, RK)],
                                out_hbm.at[cid, pl.ds(ci * RK, RK)])

    return spmm_kernel(y, col3, row3, w3)


_BLK = 1000


def _tc_linear_body(s_ref, w_ref, b_ref, o_ref):
    zz = s_ref[0] + s_ref[1]
    y = lax.dot_general(zz, w_ref[...], (((1,), (1,)), ((), ())),
                        preferred_element_type=jnp.float32,
                        precision=lax.Precision.HIGHEST) + b_ref[...]
    o_ref[...] = jnp.maximum(y, 0.0)


def _tc_final_body(s_ref, w_ref, b_ref, o_ref):
    zz = s_ref[0] + s_ref[1]
    y = lax.dot_general(zz, w_ref[...], (((1,), (1,)), ((), ())),
                        preferred_element_type=jnp.float32,
                        precision=lax.Precision.HIGHEST) + b_ref[...]
    n = jnp.sqrt(jnp.sum(y * y, axis=1, keepdims=True))
    o_ref[...] = y / jnp.maximum(n, 1e-12)


def _tc_dense(s, W, b, body):
    return pl.pallas_call(
        body,
        grid=(N_NODES // _BLK,),
        in_specs=[
            pl.BlockSpec((NC, _BLK, D), lambda i: (0, i, 0)),
            pl.BlockSpec((D, D), lambda i: (0, 0)),
            pl.BlockSpec((1, D), lambda i: (0, 0)),
        ],
        out_specs=pl.BlockSpec((_BLK, D), lambda i: (i, 0)),
        out_shape=jax.ShapeDtypeStruct((N_NODES, D), jnp.float32),
    )(s, W, b.reshape(1, D))


def kernel(x, edge_index, edge_weight, W1, b1, W2, b2, W3, b3):
    row = edge_index[0]
    col = edge_index[1]
    pad = NW * EPT - N_EDGES
    zpad_i = jnp.zeros((pad,), jnp.int32)
    colp = jnp.concatenate([col, zpad_i]).reshape(NW, NCHUNK, K)
    rowp = jnp.concatenate([row, zpad_i]).reshape(NW, NCHUNK, K)
    wp = jnp.concatenate([edge_weight, jnp.zeros((pad,), jnp.float32)]
                         ).reshape(NW, NCHUNK, K)

    s1 = _spmm_sc(x, colp, rowp, wp)
    h1 = _tc_dense(s1, W1, b1, _tc_linear_body)
    s2 = _spmm_sc(h1, colp, rowp, wp)
    h2 = _tc_dense(s2, W2, b2, _tc_linear_body)
    s3 = _spmm_sc(h2, colp, rowp, wp)
    return _tc_dense(s3, W3, b3, _tc_final_body)


# speedup vs baseline: 4.6625x; 1.2850x over previous
import dataclasses
import functools

import jax
import jax.numpy as jnp
from jax import lax
from jax.experimental import pallas as pl
from jax.experimental.pallas import tpu as pltpu
from jax.experimental.pallas import tpu_sc as plsc

N_NODES = 10000
N_EDGES = 320000
D = 128
NC = 2
NS = 16
NW = NC * NS
K = 64
NCHUNK = 158
SUB = 16
NSUB = K // SUB
EPT = NCHUNK * K
N_PAD = 10112
RK = 64
NRCHUNK = N_PAD // RK
RC_PER_TILE = 10
F16 = D // 16


def _spmm_sc(y, col3, row3, w3):
    mesh = plsc.VectorSubcoreMesh(core_axis_name="c", subcore_axis_name="s")
    cp = pltpu.CompilerParams()
    if "needs_layout_passes" in pltpu.CompilerParams.__dataclass_fields__:
        cp = dataclasses.replace(cp, needs_layout_passes=False)

    @functools.partial(
        pl.kernel,
        compiler_params=cp,
        out_type=jax.ShapeDtypeStruct((NC, N_PAD, D), jnp.float32),
        mesh=mesh,
        scratch_types=[
            pltpu.VMEM((NCHUNK // 2, 2 * K), jnp.int32),
            pltpu.VMEM((NCHUNK // 2, 2 * K), jnp.int32),
            pltpu.VMEM((NCHUNK // 2, 2 * K), jnp.float32),
            pltpu.VMEM((K, D), jnp.float32),
            pltpu.VMEM((K, D), jnp.float32),
            pltpu.VMEM_SHARED((N_PAD, D), jnp.float32),
            pltpu.SemaphoreType.DMA,
            pltpu.SemaphoreType.DMA,
        ],
    )
    def spmm_kernel(y_hbm, col_hbm, row_hbm, w_hbm, out_hbm,
                    col_v, row_v, w_v, rows0_v, rows1_v, acc, sem_g, sem_s):
        cid = lax.axis_index("c")
        sid = lax.axis_index("s")
        wid = cid * NS + sid

        @pl.loop(0, RK)
        def _(r):
            for f in range(F16):
                rows0_v[r, pl.ds(16 * f, 16)] = jnp.zeros((16,), jnp.float32)

        for i in range(RC_PER_TILE):
            ci = sid * RC_PER_TILE + i

            @pl.when(ci < NRCHUNK)
            def _():
                pltpu.sync_copy(rows0_v, acc.at[pl.ds(ci * RK, RK)])

        pltpu.sync_copy(col_hbm.at[wid], col_v)
        pltpu.sync_copy(row_hbm.at[wid], row_v)
        pltpu.sync_copy(w_hbm.at[wid], w_v)

        plsc.subcore_barrier()

        pltpu.async_copy(y_hbm.at[col_v.at[0, pl.ds(0, K)]], rows0_v, sem_g)
        bufs = (rows0_v, rows1_v)

        def _do_half(ch, b):
            buf = bufs[b]
            pltpu.make_async_copy(
                y_hbm.at[col_v.at[ch, pl.ds(b * K, K)]], buf, sem_g).wait()

            nch = ch + b
            nb = 1 - b

            @pl.when(nch < NCHUNK // 2)
            def _():
                pltpu.async_copy(
                    y_hbm.at[col_v.at[nch, pl.ds(nb * K, K)]], bufs[nb], sem_g)

            @pl.loop(0, K)
            def _(e):
                wsplat = plsc.load_gather(
                    w_v, [jnp.broadcast_to(ch, (16,)),
                          jnp.broadcast_to(b * K + e, (16,))])
                for f in range(F16):
                    sl = pl.ds(16 * f, 16)
                    buf[e, sl] = buf[e, sl] * wsplat

            for j in range(NSUB):
                idx16 = row_v[ch, pl.ds(b * K + j * SUB, SUB)]
                pltpu.sync_copy(buf.at[pl.ds(j * SUB, SUB)],
                                acc.at[idx16], add=True)

        @pl.loop(0, NCHUNK // 2)
        def _(ch):
            _do_half(ch, 0)
            _do_half(ch, 1)

        plsc.subcore_barrier()

        for i in range(RC_PER_TILE):
            ci = sid * RC_PER_TILE + i

            @pl.when(ci < NRCHUNK)
            def _():
                pltpu.sync_copy(acc.at[pl.ds(ci * RK, RK)],
                                out_hbm.at[cid, pl.ds(ci * RK, RK)])

    return spmm_kernel(y, col3, row3, w3)


_BLK = 1000


def _tc_linear_body(s_ref, w_ref, b_ref, o_ref):
    zz = s_ref[0] + s_ref[1]
    y = lax.dot_general(zz, w_ref[...], (((1,), (1,)), ((), ())),
                        preferred_element_type=jnp.float32,
                        precision=lax.Precision.HIGHEST) + b_ref[...]
    o_ref[...] = jnp.maximum(y, 0.0)


def _tc_final_body(s_ref, w_ref, b_ref, o_ref):
    zz = s_ref[0] + s_ref[1]
    y = lax.dot_general(zz, w_ref[...], (((1,), (1,)), ((), ())),
                        preferred_element_type=jnp.float32,
                        precision=lax.Precision.HIGHEST) + b_ref[...]
    n = jnp.sqrt(jnp.sum(y * y, axis=1, keepdims=True))
    o_ref[...] = y / jnp.maximum(n, 1e-12)


def _tc_dense(s, W, b, body):
    return pl.pallas_call(
        body,
        grid=(N_NODES // _BLK,),
        in_specs=[
            pl.BlockSpec((NC, _BLK, D), lambda i: (0, i, 0)),
            pl.BlockSpec((D, D), lambda i: (0, 0)),
            pl.BlockSpec((1, D), lambda i: (0, 0)),
        ],
        out_specs=pl.BlockSpec((_BLK, D), lambda i: (i, 0)),
        out_shape=jax.ShapeDtypeStruct((N_NODES, D), jnp.float32),
    )(s, W, b.reshape(1, D))


def kernel(x, edge_index, edge_weight, W1, b1, W2, b2, W3, b3):
    row = edge_index[0]
    col = edge_index[1]
    pad = NW * EPT - N_EDGES
    zpad_i = jnp.zeros((pad,), jnp.int32)
    colp = jnp.concatenate([col, zpad_i]).reshape(NW, NCHUNK // 2, 2 * K)
    rowp = jnp.concatenate([row, zpad_i]).reshape(NW, NCHUNK // 2, 2 * K)
    wp = jnp.concatenate([edge_weight, jnp.zeros((pad,), jnp.float32)]
                         ).reshape(NW, NCHUNK // 2, 2 * K)

    s1 = _spmm_sc(x, colp, rowp, wp)
    h1 = _tc_dense(s1, W1, b1, _tc_linear_body)
    s2 = _spmm_sc(h1, colp, rowp, wp)
    h2 = _tc_dense(s2, W2, b2, _tc_linear_body)
    s3 = _spmm_sc(h2, colp, rowp, wp)
    return _tc_dense(s3, W3, b3, _tc_final_body)


# speedup vs baseline: 4.9557x; 1.0629x over previous
import dataclasses
import functools

import jax
import jax.numpy as jnp
from jax import lax
from jax.experimental import pallas as pl
from jax.experimental.pallas import tpu as pltpu
from jax.experimental.pallas import tpu_sc as plsc

N_NODES = 10000
N_EDGES = 320000
D = 128
NC = 2
NS = 16
NW = NC * NS
K = 64
NCHUNK = 158
SUB = 16
NSUB = K // SUB
EPT = NCHUNK * K
N_PAD = 10112
RK = 64
NRCHUNK = N_PAD // RK
RC_PER_TILE = 10
F16 = D // 16


def _spmm_sc(y, col3, row3, w3):
    mesh = plsc.VectorSubcoreMesh(core_axis_name="c", subcore_axis_name="s")
    cp = pltpu.CompilerParams()
    if "needs_layout_passes" in pltpu.CompilerParams.__dataclass_fields__:
        cp = dataclasses.replace(cp, needs_layout_passes=False)

    @functools.partial(
        pl.kernel,
        compiler_params=cp,
        out_type=jax.ShapeDtypeStruct((NC, N_PAD, D), jnp.float32),
        mesh=mesh,
        scratch_types=[
            pltpu.VMEM((NCHUNK // 2, 2 * K), jnp.int32),
            pltpu.VMEM((NCHUNK // 2, 2 * K), jnp.int32),
            pltpu.VMEM((NCHUNK // 2, 2 * K), jnp.float32),
            pltpu.VMEM((K, D), jnp.float32),
            pltpu.VMEM((K, D), jnp.float32),
            pltpu.VMEM_SHARED((N_PAD, D), jnp.float32),
            pltpu.SemaphoreType.DMA,
            pltpu.SemaphoreType.DMA,
        ],
    )
    def spmm_kernel(y_hbm, col_hbm, row_hbm, w_hbm, out_hbm,
                    col_v, row_v, w_v, rows0_v, rows1_v, acc, sem_g, sem_s):
        cid = lax.axis_index("c")
        sid = lax.axis_index("s")
        wid = cid * NS + sid

        @pl.loop(0, RK)
        def _(r):
            for f in range(F16):
                rows0_v[r, pl.ds(16 * f, 16)] = jnp.zeros((16,), jnp.float32)

        for i in range(RC_PER_TILE):
            ci = sid * RC_PER_TILE + i

            @pl.when(ci < NRCHUNK)
            def _():
                pltpu.sync_copy(rows0_v, acc.at[pl.ds(ci * RK, RK)])

        pltpu.sync_copy(col_hbm.at[wid], col_v)
        pltpu.sync_copy(row_hbm.at[wid], row_v)
        pltpu.sync_copy(w_hbm.at[wid], w_v)

        plsc.subcore_barrier()

        pltpu.async_copy(y_hbm.at[col_v.at[0, pl.ds(0, K)]], rows0_v, sem_g)
        bufs = (rows0_v, rows1_v)

        def _do_half(ch, b):
            buf = bufs[b]
            pltpu.make_async_copy(
                y_hbm.at[col_v.at[ch, pl.ds(b * K, K)]], buf, sem_g).wait()

            nch = ch + b
            nb = 1 - b

            @pl.when(nch < NCHUNK // 2)
            def _():
                pltpu.async_copy(
                    y_hbm.at[col_v.at[nch, pl.ds(nb * K, K)]], bufs[nb], sem_g)

            @pl.loop(0, K)
            def _(e):
                wsplat = plsc.load_gather(
                    w_v, [jnp.broadcast_to(ch, (16,)),
                          jnp.broadcast_to(b * K + e, (16,))])
                for f in range(F16):
                    sl = pl.ds(16 * f, 16)
                    buf[e, sl] = buf[e, sl] * wsplat

            handles = []
            for j in range(NSUB):
                idx16 = row_v[ch, pl.ds(b * K + j * SUB, SUB)]
                handles.append(pltpu.async_copy(
                    buf.at[pl.ds(j * SUB, SUB)], acc.at[idx16], sem_s,
                    add=True))
            for h in handles:
                h.wait()

        @pl.loop(0, NCHUNK // 2)
        def _(ch):
            _do_half(ch, 0)
            _do_half(ch, 1)

        plsc.subcore_barrier()

        for i in range(RC_PER_TILE):
            ci = sid * RC_PER_TILE + i

            @pl.when(ci < NRCHUNK)
            def _():
                pltpu.sync_copy(acc.at[pl.ds(ci * RK, RK)],
                                out_hbm.at[cid, pl.ds(ci * RK, RK)])

    return spmm_kernel(y, col3, row3, w3)


_BLK = 1000


def _tc_linear_body(s_ref, w_ref, b_ref, o_ref):
    zz = s_ref[0] + s_ref[1]
    y = lax.dot_general(zz, w_ref[...], (((1,), (1,)), ((), ())),
                        preferred_element_type=jnp.float32,
                        precision=lax.Precision.HIGHEST) + b_ref[...]
    o_ref[...] = jnp.maximum(y, 0.0)


def _tc_final_body(s_ref, w_ref, b_ref, o_ref):
    zz = s_ref[0] + s_ref[1]
    y = lax.dot_general(zz, w_ref[...], (((1,), (1,)), ((), ())),
                        preferred_element_type=jnp.float32,
                        precision=lax.Precision.HIGHEST) + b_ref[...]
    n = jnp.sqrt(jnp.sum(y * y, axis=1, keepdims=True))
    o_ref[...] = y / jnp.maximum(n, 1e-12)


def _tc_dense(s, W, b, body):
    return pl.pallas_call(
        body,
        grid=(N_NODES // _BLK,),
        in_specs=[
            pl.BlockSpec((NC, _BLK, D), lambda i: (0, i, 0)),
            pl.BlockSpec((D, D), lambda i: (0, 0)),
            pl.BlockSpec((1, D), lambda i: (0, 0)),
        ],
        out_specs=pl.BlockSpec((_BLK, D), lambda i: (i, 0)),
        out_shape=jax.ShapeDtypeStruct((N_NODES, D), jnp.float32),
    )(s, W, b.reshape(1, D))


def kernel(x, edge_index, edge_weight, W1, b1, W2, b2, W3, b3):
    row = edge_index[0]
    col = edge_index[1]
    pad = NW * EPT - N_EDGES
    zpad_i = jnp.zeros((pad,), jnp.int32)
    colp = jnp.concatenate([col, zpad_i]).reshape(NW, NCHUNK // 2, 2 * K)
    rowp = jnp.concatenate([row, zpad_i]).reshape(NW, NCHUNK // 2, 2 * K)
    wp = jnp.concatenate([edge_weight, jnp.zeros((pad,), jnp.float32)]
                         ).reshape(NW, NCHUNK // 2, 2 * K)

    s1 = _spmm_sc(x, colp, rowp, wp)
    h1 = _tc_dense(s1, W1, b1, _tc_linear_body)
    s2 = _spmm_sc(h1, colp, rowp, wp)
    h2 = _tc_dense(s2, W2, b2, _tc_linear_body)
    s3 = _spmm_sc(h2, colp, rowp, wp)
    return _tc_dense(s3, W3, b3, _tc_final_body)


# speedup vs baseline: 5.0646x; 1.0220x over previous
import dataclasses
import functools

import jax
import jax.numpy as jnp
from jax import lax
from jax.experimental import pallas as pl
from jax.experimental.pallas import tpu as pltpu
from jax.experimental.pallas import tpu_sc as plsc

N_NODES = 10000
N_EDGES = 320000
D = 128
NC = 2
NS = 16
NW = NC * NS
K = 64
NCHUNK = 158
SUB = 16
NSUB = K // SUB
EPT = NCHUNK * K
N_PAD = 10112
RK = 64
NRCHUNK = N_PAD // RK
RC_PER_TILE = 10
F16 = D // 16


def _spmm_sc(y, col3, row3, w3):
    mesh = plsc.VectorSubcoreMesh(core_axis_name="c", subcore_axis_name="s")
    cp = pltpu.CompilerParams()
    if "needs_layout_passes" in pltpu.CompilerParams.__dataclass_fields__:
        cp = dataclasses.replace(cp, needs_layout_passes=False)

    @functools.partial(
        pl.kernel,
        compiler_params=cp,
        out_type=jax.ShapeDtypeStruct((NC, N_PAD, D), jnp.float32),
        mesh=mesh,
        scratch_types=[
            pltpu.VMEM((NCHUNK // 2, 2 * K), jnp.int32),
            pltpu.VMEM((NCHUNK // 2, 2 * K), jnp.int32),
            pltpu.VMEM((NCHUNK // 2, 2 * K), jnp.float32),
            pltpu.VMEM((K, D), jnp.float32),
            pltpu.VMEM((K, D), jnp.float32),
            pltpu.VMEM_SHARED((N_PAD, D), jnp.float32),
            pltpu.SemaphoreType.DMA,
            pltpu.SemaphoreType.DMA,
        ],
    )
    def spmm_kernel(y_hbm, col_hbm, row_hbm, w_hbm, out_hbm,
                    col_v, row_v, w_v, rows0_v, rows1_v, acc, sem_g, sem_s):
        cid = lax.axis_index("c")
        sid = lax.axis_index("s")
        wid = cid * NS + sid

        @pl.loop(0, RK)
        def _(r):
            for f in range(F16):
                rows0_v[r, pl.ds(16 * f, 16)] = jnp.zeros((16,), jnp.float32)

        for i in range(RC_PER_TILE):
            ci = sid * RC_PER_TILE + i

            @pl.when(ci < NRCHUNK)
            def _():
                pltpu.sync_copy(rows0_v, acc.at[pl.ds(ci * RK, RK)])

        pltpu.sync_copy(col_hbm.at[wid], col_v)
        pltpu.sync_copy(row_hbm.at[wid], row_v)
        pltpu.sync_copy(w_hbm.at[wid], w_v)

        plsc.subcore_barrier()

        pltpu.async_copy(y_hbm.at[col_v.at[0, pl.ds(0, K)]], rows0_v, sem_g)
        bufs = (rows0_v, rows1_v)

        def _do_half(ch, b):
            buf = bufs[b]
            pltpu.make_async_copy(
                y_hbm.at[col_v.at[ch, pl.ds(b * K, K)]], buf, sem_g).wait()

            nch = ch + b
            nb = 1 - b

            @pl.when(nch < NCHUNK // 2)
            def _():
                pltpu.async_copy(
                    y_hbm.at[col_v.at[nch, pl.ds(nb * K, K)]], bufs[nb], sem_g)

            @plsc.parallel_loop(0, K, unroll=4)
            def _(e):
                wsplat = plsc.load_gather(
                    w_v, [jnp.broadcast_to(ch, (16,)),
                          jnp.broadcast_to(b * K + e, (16,))])
                for f in range(F16):
                    sl = pl.ds(16 * f, 16)
                    buf[e, sl] = buf[e, sl] * wsplat

            handles = []
            for j in range(NSUB):
                idx16 = row_v[ch, pl.ds(b * K + j * SUB, SUB)]
                handles.append(pltpu.async_copy(
                    buf.at[pl.ds(j * SUB, SUB)], acc.at[idx16], sem_s,
                    add=True))
            for h in handles:
                h.wait()

        @pl.loop(0, NCHUNK // 2)
        def _(ch):
            _do_half(ch, 0)
            _do_half(ch, 1)

        plsc.subcore_barrier()

        for i in range(RC_PER_TILE):
            ci = sid * RC_PER_TILE + i

            @pl.when(ci < NRCHUNK)
            def _():
                pltpu.sync_copy(acc.at[pl.ds(ci * RK, RK)],
                                out_hbm.at[cid, pl.ds(ci * RK, RK)])

    return spmm_kernel(y, col3, row3, w3)


_BLK = 1000


def _tc_linear_body(s_ref, w_ref, b_ref, o_ref):
    zz = s_ref[0] + s_ref[1]
    y = lax.dot_general(zz, w_ref[...], (((1,), (1,)), ((), ())),
                        preferred_element_type=jnp.float32,
                        precision=lax.Precision.HIGHEST) + b_ref[...]
    o_ref[...] = jnp.maximum(y, 0.0)


def _tc_final_body(s_ref, w_ref, b_ref, o_ref):
    zz = s_ref[0] + s_ref[1]
    y = lax.dot_general(zz, w_ref[...], (((1,), (1,)), ((), ())),
                        preferred_element_type=jnp.float32,
                        precision=lax.Precision.HIGHEST) + b_ref[...]
    n = jnp.sqrt(jnp.sum(y * y, axis=1, keepdims=True))
    o_ref[...] = y / jnp.maximum(n, 1e-12)


def _tc_dense(s, W, b, body):
    return pl.pallas_call(
        body,
        grid=(N_NODES // _BLK,),
        in_specs=[
            pl.BlockSpec((NC, _BLK, D), lambda i: (0, i, 0)),
            pl.BlockSpec((D, D), lambda i: (0, 0)),
            pl.BlockSpec((1, D), lambda i: (0, 0)),
        ],
        out_specs=pl.BlockSpec((_BLK, D), lambda i: (i, 0)),
        out_shape=jax.ShapeDtypeStruct((N_NODES, D), jnp.float32),
    )(s, W, b.reshape(1, D))


def kernel(x, edge_index, edge_weight, W1, b1, W2, b2, W3, b3):
    row = edge_index[0]
    col = edge_index[1]
    pad = NW * EPT - N_EDGES
    zpad_i = jnp.zeros((pad,), jnp.int32)
    colp = jnp.concatenate([col, zpad_i]).reshape(NW, NCHUNK // 2, 2 * K)
    rowp = jnp.concatenate([row, zpad_i]).reshape(NW, NCHUNK // 2, 2 * K)
    wp = jnp.concatenate([edge_weight, jnp.zeros((pad,), jnp.float32)]
                         ).reshape(NW, NCHUNK // 2, 2 * K)

    s1 = _spmm_sc(x, colp, rowp, wp)
    h1 = _tc_dense(s1, W1, b1, _tc_linear_body)
    s2 = _spmm_sc(h1, colp, rowp, wp)
    h2 = _tc_dense(s2, W2, b2, _tc_linear_body)
    s3 = _spmm_sc(h2, colp, rowp, wp)
    return _tc_dense(s3, W3, b3, _tc_final_body)
